# Initial kernel scaffold; baseline (speedup 1.0000x reference)
import jax
import jax.numpy as jnp
from jax.experimental import pallas as pl


def kernel(x, edge_index, W1, b1, W2, b2, Wc, bc):
    raise NotImplementedError("write your pallas kernel here")


# speedup vs baseline: 30.2497x; 30.2497x over previous
import functools

import jax
import jax.numpy as jnp
from jax import lax
from jax.experimental import pallas as pl
from jax.experimental.pallas import tpu as pltpu
from jax.experimental.pallas import tpu_sc as plsc

N = 50000
NPAD = 51200
E = 800000
NC, NS = 2, 16
NW = NC * NS
CHUNK = 128
CPW = 196
EW = CPW * CHUNK
E_PAD = NW * EW
RPT = NPAD // NS
ZR = 200
F = 16
BLK = 1024


def _sc_mesh():
    return plsc.VectorSubcoreMesh(core_axis_name="c", subcore_axis_name="s",
                                  num_cores=NC, num_subcores=NS)


def _degree_body(dst_hbm, out_hbm, dst_v, ones_v, z_v, acc):
    c = lax.axis_index("c")
    s = lax.axis_index("s")
    wid = s * NC + c

    def fill(i, _):
        ones_v[pl.ds(i * 16, 16)] = jnp.full((16,), 1.0, jnp.float32)
        return 0

    lax.fori_loop(0, CHUNK // 16, fill, 0)

    def zfill(i, _):
        z_v[pl.ds(i * 16, 16)] = jnp.zeros((16,), jnp.float32)
        return 0

    lax.fori_loop(0, RPT // 16, zfill, 0)
    pltpu.sync_copy(z_v, acc.at[pl.ds(s * RPT, RPT)])
    plsc.subcore_barrier()

    pltpu.sync_copy(dst_hbm.at[wid], dst_v)

    def body(j, _):
        pltpu.sync_copy(ones_v, acc.at[dst_v.at[j]], add=True)
        return 0

    lax.fori_loop(0, CPW, body, 0)
    plsc.subcore_barrier()
    pltpu.sync_copy(acc.at[pl.ds(s * RPT, RPT)],
                    out_hbm.at[c, 0, pl.ds(s * RPT, RPT)])


@functools.cache
def _build_degree():
    return functools.partial(
        pl.kernel,
        out_type=jax.ShapeDtypeStruct((NC, 1, NPAD), jnp.float32),
        mesh=_sc_mesh(),
        scratch_types=[
            pltpu.VMEM((CPW, CHUNK), jnp.int32),
            pltpu.VMEM((CHUNK,), jnp.float32),
            pltpu.VMEM((RPT,), jnp.float32),
            pltpu.VMEM_SHARED((NPAD,), jnp.float32),
        ],
    )(_degree_body)


@functools.cache
def _build_scatter(halves):
    def scatter_kernel(*refs):
        ys = refs[:halves]
        src_hbm, dst_hbm = refs[halves], refs[halves + 1]
        outs = refs[halves + 2:2 * halves + 2]
        (src_v, dst_v, rows_a, rows_b, z_v, acc,
         sem_a, sem_b) = refs[2 * halves + 2:]
        c = lax.axis_index("c")
        s = lax.axis_index("s")
        wid = s * NC + c

        def zfill(i, _):
            z_v[i, :] = jnp.zeros((16,), jnp.float32)
            return 0

        lax.fori_loop(0, ZR, zfill, 0)
        pltpu.sync_copy(src_hbm.at[wid], src_v)
        pltpu.sync_copy(dst_hbm.at[wid], dst_v)

        for h in range(halves):
            y_hbm = ys[h]
            for t in range(RPT // ZR):
                pltpu.sync_copy(z_v, acc.at[pl.ds(s * RPT + t * ZR, ZR), :])
            plsc.subcore_barrier()

            pltpu.async_copy(y_hbm.at[src_v.at[0]], rows_a, sem_a)

            def body(j, _):
                is_even = lax.rem(j, 2) == 0

                @pl.when(is_even)
                def _():
                    pltpu.async_copy(y_hbm.at[src_v.at[j + 1]], rows_b, sem_b)
                    pltpu.make_async_copy(y_hbm.at[src_v.at[j]], rows_a,
                                          sem_a).wait()
                    pltpu.sync_copy(rows_a, acc.at[dst_v.at[j]], add=True)

                @pl.when(jnp.logical_not(is_even))
                def _():
                    pltpu.async_copy(y_hbm.at[src_v.at[j + 1]], rows_a, sem_a)
                    pltpu.make_async_copy(y_hbm.at[src_v.at[j]], rows_b,
                                          sem_b).wait()
                    pltpu.sync_copy(rows_b, acc.at[dst_v.at[j]], add=True)

                return 0

            lax.fori_loop(0, CPW - 1, body, 0)
            pltpu.make_async_copy(y_hbm.at[src_v.at[CPW - 1]], rows_b,
                                  sem_b).wait()
            pltpu.sync_copy(rows_b, acc.at[dst_v.at[CPW - 1]], add=True)

            plsc.subcore_barrier()
            pltpu.sync_copy(acc.at[pl.ds(s * RPT, RPT), :],
                            outs[h].at[c, pl.ds(s * RPT, RPT), :])

    return functools.partial(
        pl.kernel,
        out_type=[jax.ShapeDtypeStruct((NC, NPAD, F), jnp.float32)
                  for _ in range(halves)],
        mesh=_sc_mesh(),
        scratch_types=[
            pltpu.VMEM((CPW, CHUNK), jnp.int32),
            pltpu.VMEM((CPW, CHUNK), jnp.int32),
            pltpu.VMEM((CHUNK, F), jnp.float32),
            pltpu.VMEM((CHUNK, F), jnp.float32),
            pltpu.VMEM((ZR, F), jnp.float32),
            pltpu.VMEM_SHARED((NPAD, F), jnp.float32),
            pltpu.SemaphoreType.DMA,
            pltpu.SemaphoreType.DMA,
        ],
        compiler_params=pltpu.CompilerParams(use_tc_tiling_on_sc=False),
    )(scatter_kernel)


def _tc1_body(x_ref, w1a_ref, w1b_ref, d_ref, ya_ref, yb_ref, dis_ref):
    deg = d_ref[0] + d_ref[1] + 1.0
    dis = lax.rsqrt(deg)
    x = x_ref[...]
    ya_ref[...] = jnp.dot(x, w1a_ref[...],
                          preferred_element_type=jnp.float32) * dis
    yb_ref[...] = jnp.dot(x, w1b_ref[...],
                          preferred_element_type=jnp.float32) * dis
    dis_ref[...] = dis


def _tc2_body(ya_ref, yb_ref, pa_ref, pb_ref, dis_ref, b1a_ref, b1b_ref,
              w2a_ref, w2b_ref, y2_ref):
    dis = dis_ref[...]
    ha = jnp.maximum(dis * (ya_ref[...] + pa_ref[0] + pa_ref[1])
                     + b1a_ref[...], 0.0)
    hb = jnp.maximum(dis * (yb_ref[...] + pb_ref[0] + pb_ref[1])
                     + b1b_ref[...], 0.0)
    y2 = (jnp.dot(ha, w2a_ref[...], preferred_element_type=jnp.float32)
          + jnp.dot(hb, w2b_ref[...], preferred_element_type=jnp.float32))
    y2_ref[...] = y2 * dis


def _tc3_body(y2_ref, q_ref, dis_ref, b2_ref, wc_ref, bc_ref, out_ref):
    dis = dis_ref[...]
    h = jnp.maximum(dis * (y2_ref[...] + q_ref[0] + q_ref[1]) + b2_ref[...],
                    0.0)
    logits = jnp.dot(h, wc_ref[...],
                     preferred_element_type=jnp.float32) + bc_ref[...]
    out_ref[...] = jax.nn.sigmoid(logits)


def _row_spec(f):
    return pl.BlockSpec((BLK, f), lambda i: (i, 0))


def _pair_spec(f):
    return pl.BlockSpec((2, BLK, f), lambda i: (0, i, 0))


def _full_spec(shape):
    return pl.BlockSpec(shape, lambda i: tuple(0 for _ in shape))


def kernel(x, edge_index, W1, b1, W2, b2, Wc, bc):
    src = edge_index[0].astype(jnp.int32)
    dst = edge_index[1].astype(jnp.int32)
    pad = E_PAD - E
    src2 = jnp.concatenate([src, jnp.zeros((pad,), jnp.int32)])
    dst2 = jnp.concatenate([dst, jnp.full((pad,), N, jnp.int32)])
    src2 = src2.reshape(NW, CPW, CHUNK)
    dst2 = dst2.reshape(NW, CPW, CHUNK)
    xp = jnp.pad(x, ((0, NPAD - N), (0, 0)))

    degp = _build_degree()(dst2)
    degp = degp.reshape(2, NPAD, 1)

    grid = NPAD // BLK
    ya, yb, dis = pl.pallas_call(
        _tc1_body,
        grid=(grid,),
        in_specs=[_row_spec(64), _full_spec((64, 16)), _full_spec((64, 16)),
                  _pair_spec(1)],
        out_specs=[_row_spec(16), _row_spec(16), _row_spec(1)],
        out_shape=[jax.ShapeDtypeStruct((NPAD, 16), jnp.float32),
                   jax.ShapeDtypeStruct((NPAD, 16), jnp.float32),
                   jax.ShapeDtypeStruct((NPAD, 1), jnp.float32)],
    )(xp, W1[:, :16], W1[:, 16:], degp)

    pa, pb = _build_scatter(2)(ya, yb, src2, dst2)

    y2 = pl.pallas_call(
        _tc2_body,
        grid=(grid,),
        in_specs=[_row_spec(16), _row_spec(16), _pair_spec(16),
                  _pair_spec(16), _row_spec(1), _full_spec((1, 16)),
                  _full_spec((1, 16)), _full_spec((16, 16)),
                  _full_spec((16, 16))],
        out_specs=_row_spec(16),
        out_shape=jax.ShapeDtypeStruct((NPAD, 16), jnp.float32),
    )(ya, yb, pa, pb, dis, b1[:16].reshape(1, 16), b1[16:].reshape(1, 16),
      W2[:16], W2[16:])

    (q,) = _build_scatter(1)(y2, src2, dst2)

    out = pl.pallas_call(
        _tc3_body,
        grid=(grid,),
        in_specs=[_row_spec(16), _pair_spec(16), _row_spec(1),
                  _full_spec((1, 16)), _full_spec((16, 1)),
                  _full_spec((1, 1))],
        out_specs=_row_spec(1),
        out_shape=jax.ShapeDtypeStruct((NPAD, 1), jnp.float32),
    )(y2, q, dis, b2.reshape(1, 16), Wc, bc.reshape(1, 1))

    return out[:N]


# speedup vs baseline: 31.5765x; 1.0439x over previous
import functools

import jax
import jax.numpy as jnp
from jax import lax
from jax.experimental import pallas as pl
from jax.experimental.pallas import tpu as pltpu
from jax.experimental.pallas import tpu_sc as plsc

N = 50000
NPAD = 51200
E = 800000
NC, NS = 2, 16
NW = NC * NS
CHUNK = 128
CPW = 196
EW = CPW * CHUNK
E_PAD = NW * EW
RPT = NPAD // NS
ZR = 200
F = 16
BLK = 1024


def _sc_mesh():
    return plsc.VectorSubcoreMesh(core_axis_name="c", subcore_axis_name="s",
                                  num_cores=NC, num_subcores=NS)


def _degree_body(dst_hbm, out_hbm, dst_v, ones_v, z_v, acc):
    c = lax.axis_index("c")
    s = lax.axis_index("s")
    wid = s * NC + c

    def fill(i, _):
        ones_v[pl.ds(i * 16, 16)] = jnp.full((16,), 1.0, jnp.float32)
        return 0

    lax.fori_loop(0, CHUNK // 16, fill, 0)

    def zfill(i, _):
        z_v[pl.ds(i * 16, 16)] = jnp.zeros((16,), jnp.float32)
        return 0

    lax.fori_loop(0, RPT // 16, zfill, 0)
    pltpu.sync_copy(z_v, acc.at[pl.ds(s * RPT, RPT)])
    plsc.subcore_barrier()

    pltpu.sync_copy(dst_hbm.at[wid], dst_v)

    def body(j, _):
        pltpu.sync_copy(ones_v, acc.at[dst_v.at[j]], add=True)
        return 0

    lax.fori_loop(0, CPW, body, 0)
    plsc.subcore_barrier()
    pltpu.sync_copy(acc.at[pl.ds(s * RPT, RPT)],
                    out_hbm.at[c, 0, pl.ds(s * RPT, RPT)])


@functools.cache
def _build_degree():
    return functools.partial(
        pl.kernel,
        out_type=jax.ShapeDtypeStruct((NC, 1, NPAD), jnp.float32),
        mesh=_sc_mesh(),
        scratch_types=[
            pltpu.VMEM((CPW, CHUNK), jnp.int32),
            pltpu.VMEM((CHUNK,), jnp.float32),
            pltpu.VMEM((RPT,), jnp.float32),
            pltpu.VMEM_SHARED((NPAD,), jnp.float32),
        ],
    )(_degree_body)


@functools.cache
def _build_scatter(halves):
    NBUF = 4
    LOOK = 2

    def scatter_kernel(*refs):
        ys = refs[:halves]
        src_hbm, dst_hbm = refs[halves], refs[halves + 1]
        outs = refs[halves + 2:2 * halves + 2]
        rest = refs[2 * halves + 2:]
        src_v, dst_v = rest[0], rest[1]
        bufs = rest[2:2 + NBUF]
        z_v, acc = rest[2 + NBUF], rest[3 + NBUF]
        gs = rest[4 + NBUF:4 + 2 * NBUF]
        ss = rest[4 + 2 * NBUF:4 + 3 * NBUF]
        c = lax.axis_index("c")
        s = lax.axis_index("s")
        wid = s * NC + c

        def zfill(i, _):
            z_v[i, :] = jnp.zeros((16,), jnp.float32)
            return 0

        lax.fori_loop(0, ZR, zfill, 0)
        pltpu.sync_copy(src_hbm.at[wid], src_v)
        pltpu.sync_copy(dst_hbm.at[wid], dst_v)

        for h in range(halves):
            y_hbm = ys[h]
            for t in range(RPT // ZR):
                pltpu.sync_copy(z_v, acc.at[pl.ds(s * RPT + t * ZR, ZR), :])
            plsc.subcore_barrier()

            for b in range(LOOK):
                pltpu.async_copy(y_hbm.at[src_v.at[b]], bufs[b], gs[b])

            def outer(g, _):
                for b in range(NBUF):
                    j = g * NBUF + b
                    pltpu.make_async_copy(y_hbm.at[src_v.at[j]], bufs[b],
                                          gs[b]).wait()
                    pltpu.async_copy(bufs[b], acc.at[dst_v.at[j]], ss[b],
                                     add=True)
                    jn = j + LOOK
                    bn = (b + LOOK) % NBUF

                    @pl.when(jn < CPW)
                    def _():
                        @pl.when(j >= LOOK)
                        def _():
                            pltpu.make_async_copy(
                                bufs[bn], acc.at[dst_v.at[j - LOOK]],
                                ss[bn]).wait()

                        pltpu.async_copy(y_hbm.at[src_v.at[jn]], bufs[bn],
                                         gs[bn])

                return 0

            lax.fori_loop(0, CPW // NBUF, outer, 0)
            for b in range(NBUF):
                pltpu.make_async_copy(bufs[b],
                                      acc.at[dst_v.at[CPW - NBUF + b]],
                                      ss[b]).wait()

            plsc.subcore_barrier()
            pltpu.sync_copy(acc.at[pl.ds(s * RPT, RPT), :],
                            outs[h].at[c, pl.ds(s * RPT, RPT), :])

    return functools.partial(
        pl.kernel,
        out_type=[jax.ShapeDtypeStruct((NC, NPAD, F), jnp.float32)
                  for _ in range(halves)],
        mesh=_sc_mesh(),
        scratch_types=(
            [pltpu.VMEM((CPW, CHUNK), jnp.int32),
             pltpu.VMEM((CPW, CHUNK), jnp.int32)]
            + [pltpu.VMEM((CHUNK, F), jnp.float32) for _ in range(NBUF)]
            + [pltpu.VMEM((ZR, F), jnp.float32),
               pltpu.VMEM_SHARED((NPAD, F), jnp.float32)]
            + [pltpu.SemaphoreType.DMA for _ in range(2 * NBUF)]
        ),
        compiler_params=pltpu.CompilerParams(use_tc_tiling_on_sc=False),
    )(scatter_kernel)


def _tc1_body(x_ref, w1a_ref, w1b_ref, d_ref, ya_ref, yb_ref, dis_ref):
    deg = d_ref[0] + d_ref[1] + 1.0
    dis = lax.rsqrt(deg)
    x = x_ref[...]
    ya_ref[...] = jnp.dot(x, w1a_ref[...],
                          preferred_element_type=jnp.float32) * dis
    yb_ref[...] = jnp.dot(x, w1b_ref[...],
                          preferred_element_type=jnp.float32) * dis
    dis_ref[...] = dis


def _tc2_body(ya_ref, yb_ref, pa_ref, pb_ref, dis_ref, b1a_ref, b1b_ref,
              w2a_ref, w2b_ref, y2_ref):
    dis = dis_ref[...]
    ha = jnp.maximum(dis * (ya_ref[...] + pa_ref[0] + pa_ref[1])
                     + b1a_ref[...], 0.0)
    hb = jnp.maximum(dis * (yb_ref[...] + pb_ref[0] + pb_ref[1])
                     + b1b_ref[...], 0.0)
    y2 = (jnp.dot(ha, w2a_ref[...], preferred_element_type=jnp.float32)
          + jnp.dot(hb, w2b_ref[...], preferred_element_type=jnp.float32))
    y2_ref[...] = y2 * dis


def _tc3_body(y2_ref, q_ref, dis_ref, b2_ref, wc_ref, bc_ref, out_ref):
    dis = dis_ref[...]
    h = jnp.maximum(dis * (y2_ref[...] + q_ref[0] + q_ref[1]) + b2_ref[...],
                    0.0)
    logits = jnp.dot(h, wc_ref[...],
                     preferred_element_type=jnp.float32) + bc_ref[...]
    out_ref[...] = jax.nn.sigmoid(logits)


def _row_spec(f):
    return pl.BlockSpec((BLK, f), lambda i: (i, 0))


def _pair_spec(f):
    return pl.BlockSpec((2, BLK, f), lambda i: (0, i, 0))


def _full_spec(shape):
    return pl.BlockSpec(shape, lambda i: tuple(0 for _ in shape))


def kernel(x, edge_index, W1, b1, W2, b2, Wc, bc):
    src = edge_index[0].astype(jnp.int32)
    dst = edge_index[1].astype(jnp.int32)
    pad = E_PAD - E
    src2 = jnp.concatenate([src, jnp.zeros((pad,), jnp.int32)])
    dst2 = jnp.concatenate([dst, jnp.full((pad,), N, jnp.int32)])
    src2 = src2.reshape(NW, CPW, CHUNK)
    dst2 = dst2.reshape(NW, CPW, CHUNK)
    xp = jnp.pad(x, ((0, NPAD - N), (0, 0)))

    degp = _build_degree()(dst2)
    degp = degp.reshape(2, NPAD, 1)

    grid = NPAD // BLK
    ya, yb, dis = pl.pallas_call(
        _tc1_body,
        grid=(grid,),
        in_specs=[_row_spec(64), _full_spec((64, 16)), _full_spec((64, 16)),
                  _pair_spec(1)],
        out_specs=[_row_spec(16), _row_spec(16), _row_spec(1)],
        out_shape=[jax.ShapeDtypeStruct((NPAD, 16), jnp.float32),
                   jax.ShapeDtypeStruct((NPAD, 16), jnp.float32),
                   jax.ShapeDtypeStruct((NPAD, 1), jnp.float32)],
    )(xp, W1[:, :16], W1[:, 16:], degp)

    pa, pb = _build_scatter(2)(ya, yb, src2, dst2)

    y2 = pl.pallas_call(
        _tc2_body,
        grid=(grid,),
        in_specs=[_row_spec(16), _row_spec(16), _pair_spec(16),
                  _pair_spec(16), _row_spec(1), _full_spec((1, 16)),
                  _full_spec((1, 16)), _full_spec((16, 16)),
                  _full_spec((16, 16))],
        out_specs=_row_spec(16),
        out_shape=jax.ShapeDtypeStruct((NPAD, 16), jnp.float32),
    )(ya, yb, pa, pb, dis, b1[:16].reshape(1, 16), b1[16:].reshape(1, 16),
      W2[:16], W2[16:])

    (q,) = _build_scatter(1)(y2, src2, dst2)

    out = pl.pallas_call(
        _tc3_body,
        grid=(grid,),
        in_specs=[_row_spec(16), _pair_spec(16), _row_spec(1),
                  _full_spec((1, 16)), _full_spec((16, 1)),
                  _full_spec((1, 1))],
        out_specs=_row_spec(1),
        out_shape=jax.ShapeDtypeStruct((NPAD, 1), jnp.float32),
    )(y2, q, dis, b2.reshape(1, 16), Wc, bc.reshape(1, 1))

    return out[:N]


# speedup vs baseline: 34.1483x; 1.0814x over previous
import functools

import jax
import jax.numpy as jnp
from jax import lax
from jax.experimental import pallas as pl
from jax.experimental.pallas import tpu as pltpu
from jax.experimental.pallas import tpu_sc as plsc

N = 50000
NPAD = 51200
E = 800000
NC, NS = 2, 16
NW = NC * NS
CHUNK = 128
CPW = 196
EW = CPW * CHUNK
E_PAD = NW * EW
RPT = NPAD // NS
ZR = 200
F = 16
BLK = 5120
BLK3 = 5000


def _sc_mesh():
    return plsc.VectorSubcoreMesh(core_axis_name="c", subcore_axis_name="s",
                                  num_cores=NC, num_subcores=NS)


def _degree_body(dst_hbm, out_hbm, dst_v, ones_v, z_v, acc):
    c = lax.axis_index("c")
    s = lax.axis_index("s")
    wid = s * NC + c

    def fill(i, _):
        ones_v[pl.ds(i * 16, 16)] = jnp.full((16,), 1.0, jnp.float32)
        return 0

    lax.fori_loop(0, CHUNK // 16, fill, 0)

    def zfill(i, _):
        z_v[pl.ds(i * 16, 16)] = jnp.zeros((16,), jnp.float32)
        return 0

    lax.fori_loop(0, RPT // 16, zfill, 0)
    pltpu.sync_copy(z_v, acc.at[pl.ds(s * RPT, RPT)])
    plsc.subcore_barrier()

    pltpu.sync_copy(dst_hbm.at[wid], dst_v)

    def body(j, _):
        pltpu.sync_copy(ones_v, acc.at[dst_v.at[j]], add=True)
        return 0

    lax.fori_loop(0, CPW, body, 0)
    plsc.subcore_barrier()
    pltpu.sync_copy(acc.at[pl.ds(s * RPT, RPT)],
                    out_hbm.at[c, 0, pl.ds(s * RPT, RPT)])


@functools.cache
def _build_degree():
    return functools.partial(
        pl.kernel,
        out_type=jax.ShapeDtypeStruct((NC, 1, NPAD), jnp.float32),
        mesh=_sc_mesh(),
        scratch_types=[
            pltpu.VMEM((CPW, CHUNK), jnp.int32),
            pltpu.VMEM((CHUNK,), jnp.float32),
            pltpu.VMEM((RPT,), jnp.float32),
            pltpu.VMEM_SHARED((NPAD,), jnp.float32),
        ],
    )(_degree_body)


@functools.cache
def _build_scatter(halves):
    NBUF = 4
    LOOK = 2

    def scatter_kernel(*refs):
        ys = refs[:halves]
        src_hbm, dst_hbm = refs[halves], refs[halves + 1]
        outs = refs[halves + 2:2 * halves + 2]
        rest = refs[2 * halves + 2:]
        src_v, dst_v = rest[0], rest[1]
        bufs = rest[2:2 + NBUF]
        z_v, acc = rest[2 + NBUF], rest[3 + NBUF]
        gs = rest[4 + NBUF:4 + 2 * NBUF]
        ss = rest[4 + 2 * NBUF:4 + 3 * NBUF]
        c = lax.axis_index("c")
        s = lax.axis_index("s")
        wid = s * NC + c

        def zfill(i, _):
            z_v[i, :] = jnp.zeros((16,), jnp.float32)
            return 0

        lax.fori_loop(0, ZR, zfill, 0)
        pltpu.sync_copy(src_hbm.at[wid], src_v)
        pltpu.sync_copy(dst_hbm.at[wid], dst_v)

        for h in range(halves):
            y_hbm = ys[h]
            for t in range(RPT // ZR):
                pltpu.sync_copy(z_v, acc.at[pl.ds(s * RPT + t * ZR, ZR), :])
            plsc.subcore_barrier()

            for b in range(LOOK):
                pltpu.async_copy(y_hbm.at[src_v.at[b]], bufs[b], gs[b])

            def outer(g, _):
                for b in range(NBUF):
                    j = g * NBUF + b
                    pltpu.make_async_copy(y_hbm.at[src_v.at[j]], bufs[b],
                                          gs[b]).wait()
                    pltpu.async_copy(bufs[b], acc.at[dst_v.at[j]], ss[b],
                                     add=True)
                    jn = j + LOOK
                    bn = (b + LOOK) % NBUF

                    @pl.when(jn < CPW)
                    def _():
                        @pl.when(j >= LOOK)
                        def _():
                            pltpu.make_async_copy(
                                bufs[bn], acc.at[dst_v.at[j - LOOK]],
                                ss[bn]).wait()

                        pltpu.async_copy(y_hbm.at[src_v.at[jn]], bufs[bn],
                                         gs[bn])

                return 0

            lax.fori_loop(0, CPW // NBUF, outer, 0)
            for b in range(NBUF):
                pltpu.make_async_copy(bufs[b],
                                      acc.at[dst_v.at[CPW - NBUF + b]],
                                      ss[b]).wait()

            plsc.subcore_barrier()
            pltpu.sync_copy(acc.at[pl.ds(s * RPT, RPT), :],
                            outs[h].at[c, pl.ds(s * RPT, RPT), :])

    return functools.partial(
        pl.kernel,
        out_type=[jax.ShapeDtypeStruct((NC, NPAD, F), jnp.float32)
                  for _ in range(halves)],
        mesh=_sc_mesh(),
        scratch_types=(
            [pltpu.VMEM((CPW, CHUNK), jnp.int32),
             pltpu.VMEM((CPW, CHUNK), jnp.int32)]
            + [pltpu.VMEM((CHUNK, F), jnp.float32) for _ in range(NBUF)]
            + [pltpu.VMEM((ZR, F), jnp.float32),
               pltpu.VMEM_SHARED((NPAD, F), jnp.float32)]
            + [pltpu.SemaphoreType.DMA for _ in range(2 * NBUF)]
        ),
        compiler_params=pltpu.CompilerParams(use_tc_tiling_on_sc=False),
    )(scatter_kernel)


def _tc1_body(x_ref, w1a_ref, w1b_ref, d_ref, ya_ref, yb_ref, dis_ref):
    deg = d_ref[0] + d_ref[1] + 1.0
    dis = lax.rsqrt(deg)
    x = x_ref[...]
    ya_ref[...] = jnp.dot(x, w1a_ref[...],
                          preferred_element_type=jnp.float32) * dis
    yb_ref[...] = jnp.dot(x, w1b_ref[...],
                          preferred_element_type=jnp.float32) * dis
    dis_ref[...] = dis


def _tc2_body(ya_ref, yb_ref, pa_ref, pb_ref, dis_ref, b1a_ref, b1b_ref,
              w2a_ref, w2b_ref, y2_ref):
    dis = dis_ref[...]
    ha = jnp.maximum(dis * (ya_ref[...] + pa_ref[0] + pa_ref[1])
                     + b1a_ref[...], 0.0)
    hb = jnp.maximum(dis * (yb_ref[...] + pb_ref[0] + pb_ref[1])
                     + b1b_ref[...], 0.0)
    y2 = (jnp.dot(ha, w2a_ref[...], preferred_element_type=jnp.float32)
          + jnp.dot(hb, w2b_ref[...], preferred_element_type=jnp.float32))
    y2_ref[...] = y2 * dis


def _tc3_body(y2_ref, q_ref, dis_ref, b2_ref, wc_ref, bc_ref, out_ref):
    dis = dis_ref[...]
    h = jnp.maximum(dis * (y2_ref[...] + q_ref[0] + q_ref[1]) + b2_ref[...],
                    0.0)
    logits = jnp.dot(h, wc_ref[...],
                     preferred_element_type=jnp.float32) + bc_ref[...]
    out_ref[...] = jax.nn.sigmoid(logits)


def _row_spec(f, blk=BLK):
    return pl.BlockSpec((blk, f), lambda i: (i, 0))


def _pair_spec(f, blk=BLK):
    return pl.BlockSpec((2, blk, f), lambda i: (0, i, 0))


def _full_spec(shape):
    return pl.BlockSpec(shape, lambda i: tuple(0 for _ in shape))


def kernel(x, edge_index, W1, b1, W2, b2, Wc, bc):
    src = edge_index[0].astype(jnp.int32)
    dst = edge_index[1].astype(jnp.int32)
    pad = E_PAD - E
    src2 = jnp.concatenate([src, jnp.zeros((pad,), jnp.int32)])
    dst2 = jnp.concatenate([dst, jnp.full((pad,), N, jnp.int32)])
    src2 = src2.reshape(NW, CPW, CHUNK)
    dst2 = dst2.reshape(NW, CPW, CHUNK)
    xp = jnp.pad(x, ((0, NPAD - N), (0, 0)))

    degp = _build_degree()(dst2)
    degp = degp.reshape(2, NPAD, 1)

    grid = NPAD // BLK
    ya, yb, dis = pl.pallas_call(
        _tc1_body,
        grid=(grid,),
        in_specs=[_row_spec(64), _full_spec((64, 16)), _full_spec((64, 16)),
                  _pair_spec(1)],
        out_specs=[_row_spec(16), _row_spec(16), _row_spec(1)],
        out_shape=[jax.ShapeDtypeStruct((NPAD, 16), jnp.float32),
                   jax.ShapeDtypeStruct((NPAD, 16), jnp.float32),
                   jax.ShapeDtypeStruct((NPAD, 1), jnp.float32)],
    )(xp, W1[:, :16], W1[:, 16:], degp)

    pa, pb = _build_scatter(2)(ya, yb, src2, dst2)

    y2 = pl.pallas_call(
        _tc2_body,
        grid=(grid,),
        in_specs=[_row_spec(16), _row_spec(16), _pair_spec(16),
                  _pair_spec(16), _row_spec(1), _full_spec((1, 16)),
                  _full_spec((1, 16)), _full_spec((16, 16)),
                  _full_spec((16, 16))],
        out_specs=_row_spec(16),
        out_shape=jax.ShapeDtypeStruct((NPAD, 16), jnp.float32),
    )(ya, yb, pa, pb, dis, b1[:16].reshape(1, 16), b1[16:].reshape(1, 16),
      W2[:16], W2[16:])

    (q,) = _build_scatter(1)(y2, src2, dst2)

    out = pl.pallas_call(
        _tc3_body,
        grid=(N // BLK3,),
        in_specs=[_row_spec(16, BLK3), _pair_spec(16, BLK3),
                  _row_spec(1, BLK3), _full_spec((1, 16)),
                  _full_spec((16, 1)), _full_spec((1, 1))],
        out_specs=_row_spec(1, BLK3),
        out_shape=jax.ShapeDtypeStruct((N, 1), jnp.float32),
    )(y2, q, dis, b2.reshape(1, 16), Wc, bc.reshape(1, 1))

    return out


# speedup vs baseline: 35.5989x; 1.0425x over previous
import functools

import jax
import jax.numpy as jnp
from jax import lax
from jax.experimental import pallas as pl
from jax.experimental.pallas import tpu as pltpu
from jax.experimental.pallas import tpu_sc as plsc

N = 50000
NPAD = 51200
E = 800000
NC, NS = 2, 16
NW = NC * NS
CHUNK = 128
CPW = 196
EW = CPW * CHUNK
E_PAD = NW * EW
RPT = NPAD // NS
ZR = 200
F = 16
BLK = 5120
BLK3 = 5000


def _sc_mesh():
    return plsc.VectorSubcoreMesh(core_axis_name="c", subcore_axis_name="s",
                                  num_cores=NC, num_subcores=NS)


def _degree_body(dst_hbm, out_hbm, dst_v, ones_v, z_v, acc):
    c = lax.axis_index("c")
    s = lax.axis_index("s")
    wid = s * NC + c

    def fill(i, _):
        ones_v[pl.ds(i * 16, 16)] = jnp.full((16,), 1.0, jnp.float32)
        return 0

    lax.fori_loop(0, CHUNK // 16, fill, 0)

    def zfill(i, _):
        z_v[pl.ds(i * 16, 16)] = jnp.zeros((16,), jnp.float32)
        return 0

    lax.fori_loop(0, RPT // 16, zfill, 0)
    pltpu.sync_copy(z_v, acc.at[pl.ds(s * RPT, RPT)])
    plsc.subcore_barrier()

    pltpu.sync_copy(dst_hbm.at[wid], dst_v)

    def body(j, _):
        pltpu.sync_copy(ones_v, acc.at[dst_v.at[j]], add=True)
        return 0

    lax.fori_loop(0, CPW, body, 0)
    plsc.subcore_barrier()
    pltpu.sync_copy(acc.at[pl.ds(s * RPT, RPT)],
                    out_hbm.at[c, 0, pl.ds(s * RPT, RPT)])


@functools.cache
def _build_degree():
    return functools.partial(
        pl.kernel,
        out_type=jax.ShapeDtypeStruct((NC, 1, NPAD), jnp.float32),
        mesh=_sc_mesh(),
        scratch_types=[
            pltpu.VMEM((CPW, CHUNK), jnp.int32),
            pltpu.VMEM((CHUNK,), jnp.float32),
            pltpu.VMEM((RPT,), jnp.float32),
            pltpu.VMEM_SHARED((NPAD,), jnp.float32),
        ],
    )(_degree_body)


@functools.cache
def _build_scatter(halves):
    NBUF = 4
    LOOK = 2

    def scatter_kernel(*refs):
        ys = refs[:halves]
        src_hbm, dst_hbm = refs[halves], refs[halves + 1]
        outs = refs[halves + 2:2 * halves + 2]
        rest = refs[2 * halves + 2:]
        src_v, dst_v = rest[0], rest[1]
        bufs = rest[2:2 + NBUF]
        z_v, acc = rest[2 + NBUF], rest[3 + NBUF]
        gs = rest[4 + NBUF:4 + 2 * NBUF]
        ss = rest[4 + 2 * NBUF:4 + 3 * NBUF]
        c = lax.axis_index("c")
        s = lax.axis_index("s")
        wid = s * NC + c

        def zfill(i, _):
            z_v[i, :] = jnp.zeros((16,), jnp.float32)
            return 0

        lax.fori_loop(0, ZR, zfill, 0)
        pltpu.sync_copy(src_hbm.at[wid], src_v)
        pltpu.sync_copy(dst_hbm.at[wid], dst_v)

        for h in range(halves):
            y_hbm = ys[h]
            for t in range(RPT // ZR):
                pltpu.sync_copy(z_v, acc.at[pl.ds(s * RPT + t * ZR, ZR), :])
            plsc.subcore_barrier()

            for b in range(LOOK):
                pltpu.async_copy(y_hbm.at[src_v.at[b]], bufs[b], gs[b])

            def outer(g, _):
                for b in range(NBUF):
                    j = g * NBUF + b
                    pltpu.make_async_copy(y_hbm.at[src_v.at[j]], bufs[b],
                                          gs[b]).wait()
                    pltpu.async_copy(bufs[b], acc.at[dst_v.at[j]], ss[b],
                                     add=True)
                    jn = j + LOOK
                    bn = (b + LOOK) % NBUF

                    @pl.when(jn < CPW)
                    def _():
                        @pl.when(j >= LOOK)
                        def _():
                            pltpu.make_async_copy(
                                bufs[bn], acc.at[dst_v.at[j - LOOK]],
                                ss[bn]).wait()

                        pltpu.async_copy(y_hbm.at[src_v.at[jn]], bufs[bn],
                                         gs[bn])

                return 0

            lax.fori_loop(0, CPW // NBUF, outer, 0)
            for b in range(NBUF):
                pltpu.make_async_copy(bufs[b],
                                      acc.at[dst_v.at[CPW - NBUF + b]],
                                      ss[b]).wait()

            plsc.subcore_barrier()
            pltpu.sync_copy(acc.at[pl.ds(s * RPT, RPT), :],
                            outs[h].at[c, pl.ds(s * RPT, RPT), :])

    return functools.partial(
        pl.kernel,
        out_type=[jax.ShapeDtypeStruct((NC, NPAD, F), jnp.float32)
                  for _ in range(halves)],
        mesh=_sc_mesh(),
        scratch_types=(
            [pltpu.VMEM((CPW, CHUNK), jnp.int32),
             pltpu.VMEM((CPW, CHUNK), jnp.int32)]
            + [pltpu.VMEM((CHUNK, F), jnp.float32) for _ in range(NBUF)]
            + [pltpu.VMEM((ZR, F), jnp.float32),
               pltpu.VMEM_SHARED((NPAD, F), jnp.float32)]
            + [pltpu.SemaphoreType.DMA for _ in range(2 * NBUF)]
        ),
        compiler_params=pltpu.CompilerParams(use_tc_tiling_on_sc=False),
    )(scatter_kernel)


def _tc0_body(x_ref, w1_ref, z_ref):
    z_ref[...] = jnp.dot(x_ref[...], w1_ref[...],
                         preferred_element_type=jnp.float32)


def _tc1_body(z_ref, d_ref, ya_ref, yb_ref, dis_ref):
    deg = jnp.transpose(d_ref[0] + d_ref[1]) + 1.0
    dis = lax.rsqrt(deg)
    z = z_ref[...] * dis
    ya_ref[...] = z[:, :16]
    yb_ref[...] = z[:, 16:]
    dis_ref[...] = dis


def _tc2_body(ya_ref, yb_ref, pa_ref, pb_ref, dis_ref, b1a_ref, b1b_ref,
              w2a_ref, w2b_ref, y2_ref):
    dis = dis_ref[...]
    ha = jnp.maximum(dis * (ya_ref[...] + pa_ref[0] + pa_ref[1])
                     + b1a_ref[...], 0.0)
    hb = jnp.maximum(dis * (yb_ref[...] + pb_ref[0] + pb_ref[1])
                     + b1b_ref[...], 0.0)
    y2 = (jnp.dot(ha, w2a_ref[...], preferred_element_type=jnp.float32)
          + jnp.dot(hb, w2b_ref[...], preferred_element_type=jnp.float32))
    y2_ref[...] = y2 * dis


def _tc3_body(y2_ref, q_ref, dis_ref, b2_ref, wc_ref, bc_ref, out_ref):
    dis = dis_ref[...]
    h = jnp.maximum(dis * (y2_ref[...] + q_ref[0] + q_ref[1]) + b2_ref[...],
                    0.0)
    logits = jnp.dot(h, wc_ref[...],
                     preferred_element_type=jnp.float32) + bc_ref[...]
    out_ref[...] = jax.nn.sigmoid(logits)


def _row_spec(f, blk=BLK):
    return pl.BlockSpec((blk, f), lambda i: (i, 0))


def _pair_spec(f, blk=BLK):
    return pl.BlockSpec((2, blk, f), lambda i: (0, i, 0))


def _full_spec(shape):
    return pl.BlockSpec(shape, lambda i: tuple(0 for _ in shape))


def kernel(x, edge_index, W1, b1, W2, b2, Wc, bc):
    src = edge_index[0].astype(jnp.int32)
    dst = edge_index[1].astype(jnp.int32)
    pad = E_PAD - E
    src2 = jnp.concatenate([src, jnp.zeros((pad,), jnp.int32)])
    dst2 = jnp.concatenate([dst, jnp.full((pad,), N, jnp.int32)])
    src2 = src2.reshape(NW, CPW, CHUNK)
    dst2 = dst2.reshape(NW, CPW, CHUNK)
    xp = jnp.pad(x, ((0, NPAD - N), (0, 0)))

    degp = _build_degree()(dst2)

    grid = NPAD // BLK
    z = pl.pallas_call(
        _tc0_body,
        grid=(grid,),
        in_specs=[_row_spec(64), _full_spec((64, 32))],
        out_specs=_row_spec(32),
        out_shape=jax.ShapeDtypeStruct((NPAD, 32), jnp.float32),
    )(xp, W1)

    ya, yb, dis = pl.pallas_call(
        _tc1_body,
        grid=(grid,),
        in_specs=[_row_spec(32),
                  pl.BlockSpec((2, 1, BLK), lambda i: (0, 0, i))],
        out_specs=[_row_spec(16), _row_spec(16), _row_spec(1)],
        out_shape=[jax.ShapeDtypeStruct((NPAD, 16), jnp.float32),
                   jax.ShapeDtypeStruct((NPAD, 16), jnp.float32),
                   jax.ShapeDtypeStruct((NPAD, 1), jnp.float32)],
    )(z, degp)

    pa, pb = _build_scatter(2)(ya, yb, src2, dst2)

    y2 = pl.pallas_call(
        _tc2_body,
        grid=(grid,),
        in_specs=[_row_spec(16), _row_spec(16), _pair_spec(16),
                  _pair_spec(16), _row_spec(1), _full_spec((1, 16)),
                  _full_spec((1, 16)), _full_spec((16, 16)),
                  _full_spec((16, 16))],
        out_specs=_row_spec(16),
        out_shape=jax.ShapeDtypeStruct((NPAD, 16), jnp.float32),
    )(ya, yb, pa, pb, dis, b1[:16].reshape(1, 16), b1[16:].reshape(1, 16),
      W2[:16], W2[16:])

    (q,) = _build_scatter(1)(y2, src2, dst2)

    out = pl.pallas_call(
        _tc3_body,
        grid=(N // BLK3,),
        in_specs=[_row_spec(16, BLK3), _pair_spec(16, BLK3),
                  _row_spec(1, BLK3), _full_spec((1, 16)),
                  _full_spec((16, 1)), _full_spec((1, 1))],
        out_specs=_row_spec(1, BLK3),
        out_shape=jax.ShapeDtypeStruct((N, 1), jnp.float32),
    )(y2, q, dis, b2.reshape(1, 16), Wc, bc.reshape(1, 1))

    return out
